# Initial kernel scaffold; baseline (speedup 1.0000x reference)
import jax
import jax.numpy as jnp
from jax.experimental import pallas as pl


def kernel(x, edge_index, W, att_src, att_dst, bias, gamma, beta):
    raise NotImplementedError("write your pallas kernel here")


# speedup vs baseline: 42.4128x; 42.4128x over previous
import functools

import jax
import jax.numpy as jnp
from jax import lax
from jax.experimental import pallas as pl
from jax.experimental.pallas import tpu as pltpu
from jax.experimental.pallas import tpu_sc as plsc

N_NODES = 10000
N_EDGES = 320000
FEAT_IN = 128
N_HEADS = 8
FEAT_OUT = 16
SLOPE = 0.2

HROW = 80
N_PAD = 10112
E_TOT = N_EDGES + N_NODES
EB = 128
CPT = 20736
BPT = CPT // EB
E_PAD = 16 * CPT
RPT = N_PAD // 16


def _proj_body(x_ref, w_ref, ab_ref, h_ref, asd_ref, m_ref):
    h = jnp.dot(x_ref[...], w_ref[...], preferred_element_type=jnp.float32)
    h_ref[...] = h
    asd = jnp.dot(h, ab_ref[...], preferred_element_type=jnp.float32)
    asd_ref[...] = asd
    m = jnp.broadcast_to(jnp.max(asd, axis=0, keepdims=True), (8, 128))

    @pl.when(pl.program_id(0) == 0)
    def _():
        m_ref[...] = m

    @pl.when(pl.program_id(0) != 0)
    def _():
        m_ref[...] = jnp.maximum(m_ref[...], m)


def _final_body(p_ref, d_ref, b_ref, g_ref, be_ref, o_ref):
    o = p_ref[...] / (d_ref[...] + 1e-16) + b_ref[...]
    mean = jnp.mean(o, axis=0, keepdims=True)
    c = o - mean
    var = jnp.mean(c * c, axis=0, keepdims=True)
    o = c / jnp.sqrt(var + 1e-5) * g_ref[...] + be_ref[...]
    o_ref[...] = jnp.maximum(o, SLOPE * o)


_SC_MESH = plsc.VectorSubcoreMesh(core_axis_name="c", subcore_axis_name="s")


@functools.partial(
    pl.kernel,
    mesh=_SC_MESH,
    out_type=jax.ShapeDtypeStruct((2, N_PAD, HROW), jnp.float32),
    compiler_params=pltpu.CompilerParams(use_tc_tiling_on_sc=False),
    scratch_types=[
        pltpu.VMEM((EB,), jnp.int32),
        pltpu.VMEM((EB,), jnp.int32),
        pltpu.VMEM((EB, HROW), jnp.float32),
        pltpu.VMEM((EB, HROW), jnp.float32),
        pltpu.VMEM((EB, HROW), jnp.float32),
        pltpu.VMEM((HROW,), jnp.float32),
        pltpu.VMEM_SHARED((N_PAD, HROW), jnp.float32),
        pltpu.SemaphoreType.DMA,
        pltpu.SemaphoreType.DMA,
        pltpu.SemaphoreType.DMA,
    ],
)
def _edge_kernel(as0, ad0, h0, b0, as1, ad1, h1, b1, src_hbm, dst_hbm,
                 out_hbm, src_v, dst_v, s_v, d_v, h_v, b_v, acc_sh,
                 sem_s, sem_d, sem_h):
    cid = lax.axis_index("c")
    sid = lax.axis_index("s")

    zeros16 = jnp.zeros((16,), jnp.float32)

    def zero_row(r, carry):
        for j in range(HROW // 16):
            h_v[r, pl.ds(j * 16, 16)] = zeros16
        return carry

    lax.fori_loop(0, EB, zero_row, 0)

    for k in range(5):
        sz = EB if k < 4 else RPT - 4 * EB
        rb = sid * RPT + k * EB
        pltpu.sync_copy(h_v.at[pl.ds(0, sz)], acc_sh.at[pl.ds(rb, sz)])
    plsc.subcore_barrier()

    def run_half(as_t, ad_t, h_t, b_t):
        pltpu.sync_copy(b_t, b_v)
        bcols = [b_v[pl.ds(j * 16, 16)] for j in range(HROW // 16)]

        def block_body(b, carry):
            base = pl.multiple_of(sid * CPT + b * EB, EB)
            pltpu.sync_copy(src_hbm.at[pl.ds(base, EB)], src_v)
            pltpu.sync_copy(dst_hbm.at[pl.ds(base, EB)], dst_v)
            cp_s = pltpu.async_copy(as_t.at[src_v], s_v, sem_s)
            cp_d = pltpu.async_copy(ad_t.at[dst_v], d_v, sem_d)
            cp_h = pltpu.async_copy(h_t.at[src_v], h_v, sem_h)
            cp_s.wait()
            cp_d.wait()
            cp_h.wait()

            def row_body(r, inner):
                for j in range(HROW // 16):
                    sl = pl.ds(j * 16, 16)
                    a = s_v[r, sl] + d_v[r, sl]
                    a = jnp.maximum(a, SLOPE * a)
                    e = jnp.exp(a - bcols[j])
                    h_v[r, sl] = h_v[r, sl] * e
                return inner

            lax.fori_loop(0, EB, row_body, 0)
            pltpu.sync_copy(h_v, acc_sh.at[dst_v], add=True)
            return carry

        lax.fori_loop(0, BPT, block_body, 0)

    @pl.when(cid == 0)
    def _():
        run_half(as0, ad0, h0, b0)

    @pl.when(cid == 1)
    def _():
        run_half(as1, ad1, h1, b1)

    plsc.subcore_barrier()
    for k in range(5):
        sz = EB if k < 4 else RPT - 4 * EB
        rb = sid * RPT + k * EB
        pltpu.sync_copy(acc_sh.at[pl.ds(rb, sz)],
                        out_hbm.at[cid, pl.ds(rb, sz)])


def kernel(x, edge_index, W, att_src, att_dst, bias, gamma, beta):
    f32 = jnp.float32
    eye8 = jnp.eye(N_HEADS, dtype=f32)
    blk_s = (att_src[0][:, :, None] * eye8[:, None, :]).reshape(FEAT_IN, N_HEADS)
    blk_d = (att_dst[0][:, :, None] * eye8[:, None, :]).reshape(FEAT_IN, N_HEADS)
    ab = jnp.concatenate(
        [blk_s, blk_d, jnp.zeros((FEAT_IN, FEAT_IN - 2 * N_HEADS), f32)], axis=1)

    h, asd, m8 = pl.pallas_call(
        _proj_body,
        grid=(10,),
        in_specs=[
            pl.BlockSpec((1000, FEAT_IN), lambda i: (i, 0)),
            pl.BlockSpec((FEAT_IN, FEAT_IN), lambda i: (0, 0)),
            pl.BlockSpec((FEAT_IN, FEAT_IN), lambda i: (0, 0)),
        ],
        out_specs=[
            pl.BlockSpec((1000, FEAT_IN), lambda i: (i, 0)),
            pl.BlockSpec((1000, FEAT_IN), lambda i: (i, 0)),
            pl.BlockSpec((8, FEAT_IN), lambda i: (0, 0)),
        ],
        out_shape=[
            jax.ShapeDtypeStruct((N_NODES, FEAT_IN), f32),
            jax.ShapeDtypeStruct((N_NODES, FEAT_IN), f32),
            jax.ShapeDtypeStruct((8, FEAT_IN), f32),
        ],
    )(x, W, ab)

    m16 = m8[0, :2 * N_HEADS]
    bsum = m16[:N_HEADS] + m16[N_HEADS:]
    b8 = jnp.maximum(bsum, SLOPE * bsum)

    npad = N_PAD - N_NODES
    h3 = h.reshape(N_NODES, N_HEADS, FEAT_OUT)
    rowpad = jnp.zeros((npad, HROW), f32)
    colpad8 = jnp.zeros((N_NODES, 8), f32)

    halves = []
    for c in (0, 1):
        hs = slice(4 * c, 4 * c + 4)
        as_h = jnp.repeat(asd[:, :N_HEADS][:, hs], 18, axis=1)
        ad_h = jnp.repeat(asd[:, N_HEADS:2 * N_HEADS][:, hs], 18, axis=1)
        as_h = jnp.concatenate([as_h, colpad8], axis=1)
        ad_h = jnp.concatenate([ad_h, colpad8], axis=1)
        as_h = jnp.concatenate([as_h, rowpad], axis=0)
        ad_h = jnp.concatenate([ad_h, rowpad], axis=0)
        h_h = jnp.concatenate(
            [h3[:, hs], jnp.ones((N_NODES, 4, 1), f32),
             jnp.zeros((N_NODES, 4, 1), f32)], axis=2).reshape(N_NODES, 72)
        h_h = jnp.concatenate([h_h, colpad8], axis=1)
        h_h = jnp.concatenate([h_h, rowpad], axis=0)
        b_h = jnp.concatenate([jnp.repeat(b8[hs], 18), jnp.zeros((8,), f32)])
        halves += [as_h, ad_h, h_h, b_h]

    loop = jnp.arange(N_NODES, dtype=edge_index.dtype)
    fill = jnp.full((E_PAD - E_TOT,), N_NODES, dtype=edge_index.dtype)
    src_i = jnp.concatenate([edge_index[0], loop, fill]).astype(jnp.int32)
    dst_i = jnp.concatenate([edge_index[1], loop, fill]).astype(jnp.int32)

    part = _edge_kernel(*halves, src_i, dst_i)

    out4 = part[:, :N_NODES, :72].reshape(2, N_NODES, 4, 18)
    feats = jnp.concatenate([out4[0, ..., :FEAT_OUT], out4[1, ..., :FEAT_OUT]],
                            axis=1).reshape(N_NODES, N_HEADS * FEAT_OUT)
    den = jnp.concatenate([out4[0, ..., FEAT_OUT], out4[1, ..., FEAT_OUT]],
                          axis=1)
    den = jnp.repeat(den, FEAT_OUT, axis=-1)

    return pl.pallas_call(
        _final_body,
        out_shape=jax.ShapeDtypeStruct((N_NODES, N_HEADS * FEAT_OUT), f32),
    )(feats, den, bias.reshape(1, -1), gamma.reshape(1, -1),
      beta.reshape(1, -1))


# speedup vs baseline: 59.7450x; 1.4087x over previous
import functools

import jax
import jax.numpy as jnp
from jax import lax
from jax.experimental import pallas as pl
from jax.experimental.pallas import tpu as pltpu
from jax.experimental.pallas import tpu_sc as plsc

N_NODES = 10000
N_EDGES = 320000
FEAT_IN = 128
N_HEADS = 8
FEAT_OUT = 16
SLOPE = 0.2

HROW = 80
N_PAD = 10112
E_TOT = N_EDGES + N_NODES
EB = 128
CPT = 20736
BPT = CPT // EB
E_PAD = 16 * CPT
RPT = N_PAD // 16


def _proj_body(x_ref, w_ref, ab_ref, h_ref, asd_ref, m_ref):
    h = jnp.dot(x_ref[...], w_ref[...], preferred_element_type=jnp.float32)
    h_ref[...] = h
    asd = jnp.dot(h, ab_ref[...], preferred_element_type=jnp.float32)
    asd_ref[...] = asd
    m = jnp.broadcast_to(jnp.max(asd, axis=0, keepdims=True), (8, 128))

    @pl.when(pl.program_id(0) == 0)
    def _():
        m_ref[...] = m

    @pl.when(pl.program_id(0) != 0)
    def _():
        m_ref[...] = jnp.maximum(m_ref[...], m)


def _final_body(p_ref, d_ref, b_ref, g_ref, be_ref, o_ref):
    o = p_ref[...] / (d_ref[...] + 1e-16) + b_ref[...]
    mean = jnp.mean(o, axis=0, keepdims=True)
    c = o - mean
    var = jnp.mean(c * c, axis=0, keepdims=True)
    o = c / jnp.sqrt(var + 1e-5) * g_ref[...] + be_ref[...]
    o_ref[...] = jnp.maximum(o, SLOPE * o)


_SC_MESH = plsc.VectorSubcoreMesh(core_axis_name="c", subcore_axis_name="s")


@functools.partial(
    pl.kernel,
    mesh=_SC_MESH,
    out_type=jax.ShapeDtypeStruct((2, N_PAD, HROW), jnp.float32),
    compiler_params=pltpu.CompilerParams(use_tc_tiling_on_sc=False),
    scratch_types=[
        pltpu.VMEM((EB,), jnp.int32),
        pltpu.VMEM((EB,), jnp.int32),
        pltpu.VMEM((EB,), jnp.int32),
        pltpu.VMEM((EB,), jnp.int32),
        pltpu.VMEM((EB, HROW), jnp.float32),
        pltpu.VMEM((EB, HROW), jnp.float32),
        pltpu.VMEM((EB, HROW), jnp.float32),
        pltpu.VMEM((EB, HROW), jnp.float32),
        pltpu.VMEM((EB, HROW), jnp.float32),
        pltpu.VMEM((EB, HROW), jnp.float32),
        pltpu.VMEM((HROW,), jnp.float32),
        pltpu.VMEM_SHARED((N_PAD, HROW), jnp.float32),
        pltpu.SemaphoreType.DMA,
        pltpu.SemaphoreType.DMA,
    ],
)
def _edge_kernel(as0, ad0, h0, b0, as1, ad1, h1, b1, src_hbm, dst_hbm,
                 out_hbm, sv0, dv0, sv1, dv1, s_v0, d_v0, h_v0,
                 s_v1, d_v1, h_v1, b_v, acc_sh, sem0, sem1):
    cid = lax.axis_index("c")
    sid = lax.axis_index("s")

    zeros16 = jnp.zeros((16,), jnp.float32)

    def zero_row(r, carry):
        for j in range(HROW // 16):
            h_v0[r, pl.ds(j * 16, 16)] = zeros16
        return carry

    lax.fori_loop(0, EB, zero_row, 0)

    for k in range(5):
        sz = EB if k < 4 else RPT - 4 * EB
        rb = sid * RPT + k * EB
        pltpu.sync_copy(h_v0.at[pl.ds(0, sz)], acc_sh.at[pl.ds(rb, sz)])
    plsc.subcore_barrier()

    def run_half(as_t, ad_t, h_t, b_t):
        pltpu.sync_copy(b_t, b_v)
        bcols = [b_v[pl.ds(j * 16, 16)] for j in range(HROW // 16)]
        bufs = ((sv0, dv0, s_v0, d_v0, h_v0, sem0),
                (sv1, dv1, s_v1, d_v1, h_v1, sem1))

        def issue(b, buf):
            sv, dv, s_v, d_v, h_v, sem = buf
            base = pl.multiple_of(sid * CPT + b * EB, EB)
            pltpu.sync_copy(src_hbm.at[pl.ds(base, EB)], sv)
            pltpu.sync_copy(dst_hbm.at[pl.ds(base, EB)], dv)
            pltpu.async_copy(as_t.at[sv], s_v, sem)
            pltpu.async_copy(ad_t.at[dv], d_v, sem)
            pltpu.async_copy(h_t.at[sv], h_v, sem)

        def drain_compute_scatter(buf):
            sv, dv, s_v, d_v, h_v, sem = buf
            pltpu.make_async_copy(as_t.at[sv], s_v, sem).wait()
            pltpu.make_async_copy(ad_t.at[dv], d_v, sem).wait()
            pltpu.make_async_copy(h_t.at[sv], h_v, sem).wait()

            def row_body(r, inner):
                for j in range(HROW // 16):
                    sl = pl.ds(j * 16, 16)
                    a = s_v[r, sl] + d_v[r, sl]
                    a = jnp.maximum(a, SLOPE * a)
                    e = jnp.exp(a - bcols[j])
                    h_v[r, sl] = h_v[r, sl] * e
                return inner

            lax.fori_loop(0, EB, row_body, 0)
            pltpu.sync_copy(h_v, acc_sh.at[dv], add=True)

        issue(0, bufs[0])

        def block_pair(i, carry):
            issue(2 * i + 1, bufs[1])
            drain_compute_scatter(bufs[0])

            @pl.when(i < BPT // 2 - 1)
            def _():
                issue(2 * i + 2, bufs[0])

            drain_compute_scatter(bufs[1])
            return carry

        lax.fori_loop(0, BPT // 2, block_pair, 0)

    @pl.when(cid == 0)
    def _():
        run_half(as0, ad0, h0, b0)

    @pl.when(cid == 1)
    def _():
        run_half(as1, ad1, h1, b1)

    plsc.subcore_barrier()
    for k in range(5):
        sz = EB if k < 4 else RPT - 4 * EB
        rb = sid * RPT + k * EB
        pltpu.sync_copy(acc_sh.at[pl.ds(rb, sz)],
                        out_hbm.at[cid, pl.ds(rb, sz)])


def kernel(x, edge_index, W, att_src, att_dst, bias, gamma, beta):
    f32 = jnp.float32
    eye8 = jnp.eye(N_HEADS, dtype=f32)
    blk_s = (att_src[0][:, :, None] * eye8[:, None, :]).reshape(FEAT_IN, N_HEADS)
    blk_d = (att_dst[0][:, :, None] * eye8[:, None, :]).reshape(FEAT_IN, N_HEADS)
    ab = jnp.concatenate(
        [blk_s, blk_d, jnp.zeros((FEAT_IN, FEAT_IN - 2 * N_HEADS), f32)], axis=1)

    h, asd, m8 = pl.pallas_call(
        _proj_body,
        grid=(10,),
        in_specs=[
            pl.BlockSpec((1000, FEAT_IN), lambda i: (i, 0)),
            pl.BlockSpec((FEAT_IN, FEAT_IN), lambda i: (0, 0)),
            pl.BlockSpec((FEAT_IN, FEAT_IN), lambda i: (0, 0)),
        ],
        out_specs=[
            pl.BlockSpec((1000, FEAT_IN), lambda i: (i, 0)),
            pl.BlockSpec((1000, FEAT_IN), lambda i: (i, 0)),
            pl.BlockSpec((8, FEAT_IN), lambda i: (0, 0)),
        ],
        out_shape=[
            jax.ShapeDtypeStruct((N_NODES, FEAT_IN), f32),
            jax.ShapeDtypeStruct((N_NODES, FEAT_IN), f32),
            jax.ShapeDtypeStruct((8, FEAT_IN), f32),
        ],
    )(x, W, ab)

    m16 = m8[0, :2 * N_HEADS]
    bsum = m16[:N_HEADS] + m16[N_HEADS:]
    b8 = jnp.maximum(bsum, SLOPE * bsum)

    npad = N_PAD - N_NODES
    h3 = h.reshape(N_NODES, N_HEADS, FEAT_OUT)
    rowpad = jnp.zeros((npad, HROW), f32)
    colpad8 = jnp.zeros((N_NODES, 8), f32)

    halves = []
    for c in (0, 1):
        hs = slice(4 * c, 4 * c + 4)
        as_h = jnp.repeat(asd[:, :N_HEADS][:, hs], 18, axis=1)
        ad_h = jnp.repeat(asd[:, N_HEADS:2 * N_HEADS][:, hs], 18, axis=1)
        as_h = jnp.concatenate([as_h, colpad8], axis=1)
        ad_h = jnp.concatenate([ad_h, colpad8], axis=1)
        as_h = jnp.concatenate([as_h, rowpad], axis=0)
        ad_h = jnp.concatenate([ad_h, rowpad], axis=0)
        h_h = jnp.concatenate(
            [h3[:, hs], jnp.ones((N_NODES, 4, 1), f32),
             jnp.zeros((N_NODES, 4, 1), f32)], axis=2).reshape(N_NODES, 72)
        h_h = jnp.concatenate([h_h, colpad8], axis=1)
        h_h = jnp.concatenate([h_h, rowpad], axis=0)
        b_h = jnp.concatenate([jnp.repeat(b8[hs], 18), jnp.zeros((8,), f32)])
        halves += [as_h, ad_h, h_h, b_h]

    loop = jnp.arange(N_NODES, dtype=edge_index.dtype)
    fill = jnp.full((E_PAD - E_TOT,), N_NODES, dtype=edge_index.dtype)
    src_i = jnp.concatenate([edge_index[0], loop, fill]).astype(jnp.int32)
    dst_i = jnp.concatenate([edge_index[1], loop, fill]).astype(jnp.int32)

    part = _edge_kernel(*halves, src_i, dst_i)

    out4 = part[:, :N_NODES, :72].reshape(2, N_NODES, 4, 18)
    feats = jnp.concatenate([out4[0, ..., :FEAT_OUT], out4[1, ..., :FEAT_OUT]],
                            axis=1).reshape(N_NODES, N_HEADS * FEAT_OUT)
    den = jnp.concatenate([out4[0, ..., FEAT_OUT], out4[1, ..., FEAT_OUT]],
                          axis=1)
    den = jnp.repeat(den, FEAT_OUT, axis=-1)

    return pl.pallas_call(
        _final_body,
        out_shape=jax.ShapeDtypeStruct((N_NODES, N_HEADS * FEAT_OUT), f32),
    )(feats, den, bias.reshape(1, -1), gamma.reshape(1, -1),
      beta.reshape(1, -1))


# speedup vs baseline: 71.3253x; 1.1938x over previous
import functools

import jax
import jax.numpy as jnp
import numpy as np
from jax import lax
from jax.experimental import pallas as pl
from jax.experimental.pallas import tpu as pltpu
from jax.experimental.pallas import tpu_sc as plsc

N_NODES = 10000
N_EDGES = 320000
FEAT_IN = 128
N_HEADS = 8
FEAT_OUT = 16
SLOPE = 0.2

HROW = 80
N_PAD = 10112
E_TOT = N_EDGES + N_NODES
EB = 128
CPT = 20736
BPT = CPT // EB
E_PAD = 16 * CPT
RPT = N_PAD // 16


def _build_consts():
    ms = np.zeros((2, FEAT_IN, HROW), np.float32)
    md = np.zeros((2, FEAT_IN, HROW), np.float32)
    pc = np.zeros((2, FEAT_IN, HROW), np.float32)
    one = np.zeros((1, HROW), np.float32)
    fm = np.zeros((HROW, 64), np.float32)
    dm = np.zeros((HROW, 64), np.float32)
    for k in range(4):
        one[0, 18 * k + 16] = 1.0
        for f in range(FEAT_OUT):
            fm[18 * k + f, 16 * k + f] = 1.0
            dm[18 * k + 16, 16 * k + f] = 1.0
    for c in range(2):
        for k in range(4):
            head = 4 * c + k
            for t in range(18):
                ms[c, head, 18 * k + t] = 1.0
                md[c, N_HEADS + head, 18 * k + t] = 1.0
            for f in range(FEAT_OUT):
                pc[c, FEAT_OUT * head + f, 18 * k + f] = 1.0
    return ms, md, pc, one, fm, dm


_MS, _MD, _PC, _ONE, _FM, _DM = _build_consts()


def _proj_body(x_ref, w_ref, ab_ref, ms_ref, md_ref, pc_ref, one_ref,
               s0_ref, d0_ref, h0_ref, s1_ref, d1_ref, h1_ref, m_ref):
    h = jnp.dot(x_ref[...], w_ref[...], preferred_element_type=jnp.float32)
    asd = jnp.dot(h, ab_ref[...], preferred_element_type=jnp.float32)
    for c, (s_ref, d_ref, hh_ref) in enumerate(
            ((s0_ref, d0_ref, h0_ref), (s1_ref, d1_ref, h1_ref))):
        s_ref[...] = jnp.dot(asd, ms_ref[c],
                             preferred_element_type=jnp.float32)
        d_ref[...] = jnp.dot(asd, md_ref[c],
                             preferred_element_type=jnp.float32)
        hh_ref[...] = jnp.dot(h, pc_ref[c],
                              preferred_element_type=jnp.float32) + one_ref[...]
    m = jnp.broadcast_to(jnp.max(asd, axis=0, keepdims=True), (8, 128))

    @pl.when(pl.program_id(0) == 0)
    def _():
        m_ref[...] = m

    @pl.when(pl.program_id(0) != 0)
    def _():
        m_ref[...] = jnp.maximum(m_ref[...], m)


def _final_body(p_ref, fm_ref, dm_ref, b_ref, g_ref, be_ref, o_ref):
    p0 = p_ref[0, :N_NODES, :]
    p1 = p_ref[1, :N_NODES, :]
    fm = fm_ref[...]
    dm = dm_ref[...]
    feats = jnp.concatenate(
        [jnp.dot(p0, fm, preferred_element_type=jnp.float32),
         jnp.dot(p1, fm, preferred_element_type=jnp.float32)], axis=1)
    den = jnp.concatenate(
        [jnp.dot(p0, dm, preferred_element_type=jnp.float32),
         jnp.dot(p1, dm, preferred_element_type=jnp.float32)], axis=1)
    o = feats / (den + 1e-16) + b_ref[...]
    mean = jnp.mean(o, axis=0, keepdims=True)
    c = o - mean
    var = jnp.mean(c * c, axis=0, keepdims=True)
    o = c / jnp.sqrt(var + 1e-5) * g_ref[...] + be_ref[...]
    o_ref[...] = jnp.maximum(o, SLOPE * o)


_SC_MESH = plsc.VectorSubcoreMesh(core_axis_name="c", subcore_axis_name="s")


@functools.partial(
    pl.kernel,
    mesh=_SC_MESH,
    out_type=jax.ShapeDtypeStruct((2, N_PAD, HROW), jnp.float32),
    compiler_params=pltpu.CompilerParams(use_tc_tiling_on_sc=False),
    scratch_types=[
        pltpu.VMEM((EB,), jnp.int32),
        pltpu.VMEM((EB,), jnp.int32),
        pltpu.VMEM((EB,), jnp.int32),
        pltpu.VMEM((EB,), jnp.int32),
        pltpu.VMEM((EB, HROW), jnp.float32),
        pltpu.VMEM((EB, HROW), jnp.float32),
        pltpu.VMEM((EB, HROW), jnp.float32),
        pltpu.VMEM((EB, HROW), jnp.float32),
        pltpu.VMEM((EB, HROW), jnp.float32),
        pltpu.VMEM((EB, HROW), jnp.float32),
        pltpu.VMEM((HROW,), jnp.float32),
        pltpu.VMEM_SHARED((N_PAD, HROW), jnp.float32),
        pltpu.SemaphoreType.DMA,
        pltpu.SemaphoreType.DMA,
    ],
)
def _edge_kernel(as0, ad0, h0, b0, as1, ad1, h1, b1, src_hbm, dst_hbm,
                 out_hbm, sv0, dv0, sv1, dv1, s_v0, d_v0, h_v0,
                 s_v1, d_v1, h_v1, b_v, acc_sh, sem0, sem1):
    cid = lax.axis_index("c")
    sid = lax.axis_index("s")

    zeros16 = jnp.zeros((16,), jnp.float32)

    def zero_row(r, carry):
        for j in range(HROW // 16):
            h_v0[r, pl.ds(j * 16, 16)] = zeros16
        return carry

    lax.fori_loop(0, EB, zero_row, 0)

    for k in range(5):
        sz = EB if k < 4 else RPT - 4 * EB
        rb = sid * RPT + k * EB
        pltpu.sync_copy(h_v0.at[pl.ds(0, sz)], acc_sh.at[pl.ds(rb, sz)])
    plsc.subcore_barrier()

    def run_half(as_t, ad_t, h_t, b_t):
        pltpu.sync_copy(b_t, b_v)
        bcols = [b_v[pl.ds(j * 16, 16)] for j in range(HROW // 16)]
        bufs = ((sv0, dv0, s_v0, d_v0, h_v0, sem0),
                (sv1, dv1, s_v1, d_v1, h_v1, sem1))

        def issue(b, buf):
            sv, dv, s_v, d_v, h_v, sem = buf
            base = pl.multiple_of(sid * CPT + b * EB, EB)
            pltpu.sync_copy(src_hbm.at[pl.ds(base, EB)], sv)
            pltpu.sync_copy(dst_hbm.at[pl.ds(base, EB)], dv)
            pltpu.async_copy(as_t.at[sv], s_v, sem)
            pltpu.async_copy(ad_t.at[dv], d_v, sem)
            pltpu.async_copy(h_t.at[sv], h_v, sem)

        def drain_compute_scatter(buf):
            sv, dv, s_v, d_v, h_v, sem = buf
            pltpu.make_async_copy(as_t.at[sv], s_v, sem).wait()
            pltpu.make_async_copy(ad_t.at[dv], d_v, sem).wait()
            pltpu.make_async_copy(h_t.at[sv], h_v, sem).wait()

            def row_body(r, inner):
                for j in range(HROW // 16):
                    sl = pl.ds(j * 16, 16)
                    a = s_v[r, sl] + d_v[r, sl]
                    a = jnp.maximum(a, SLOPE * a)
                    e = jnp.exp(a - bcols[j])
                    h_v[r, sl] = h_v[r, sl] * e
                return inner

            lax.fori_loop(0, EB, row_body, 0)
            pltpu.sync_copy(h_v, acc_sh.at[dv], add=True)

        issue(0, bufs[0])

        def block_pair(i, carry):
            issue(2 * i + 1, bufs[1])
            drain_compute_scatter(bufs[0])

            @pl.when(i < BPT // 2 - 1)
            def _():
                issue(2 * i + 2, bufs[0])

            drain_compute_scatter(bufs[1])
            return carry

        lax.fori_loop(0, BPT // 2, block_pair, 0)

    @pl.when(cid == 0)
    def _():
        run_half(as0, ad0, h0, b0)

    @pl.when(cid == 1)
    def _():
        run_half(as1, ad1, h1, b1)

    plsc.subcore_barrier()
    for k in range(5):
        sz = EB if k < 4 else RPT - 4 * EB
        rb = sid * RPT + k * EB
        pltpu.sync_copy(acc_sh.at[pl.ds(rb, sz)],
                        out_hbm.at[cid, pl.ds(rb, sz)])


def kernel(x, edge_index, W, att_src, att_dst, bias, gamma, beta):
    f32 = jnp.float32
    eye8 = jnp.eye(N_HEADS, dtype=f32)
    blk_s = (att_src[0][:, :, None] * eye8[:, None, :]).reshape(FEAT_IN, N_HEADS)
    blk_d = (att_dst[0][:, :, None] * eye8[:, None, :]).reshape(FEAT_IN, N_HEADS)
    ab = jnp.concatenate(
        [blk_s, blk_d, jnp.zeros((FEAT_IN, FEAT_IN - 2 * N_HEADS), f32)], axis=1)

    x_pad = jnp.concatenate(
        [x, jnp.zeros((N_PAD - N_NODES, FEAT_IN), f32)], axis=0)

    tblk = N_PAD // 16
    table_spec = pl.BlockSpec((tblk, HROW), lambda i: (i, 0))
    table_shape = jax.ShapeDtypeStruct((N_PAD, HROW), f32)
    s0, d0, h0, s1, d1, h1, m8 = pl.pallas_call(
        _proj_body,
        grid=(16,),
        in_specs=[
            pl.BlockSpec((tblk, FEAT_IN), lambda i: (i, 0)),
            pl.BlockSpec((FEAT_IN, FEAT_IN), lambda i: (0, 0)),
            pl.BlockSpec((FEAT_IN, FEAT_IN), lambda i: (0, 0)),
            pl.BlockSpec((2, FEAT_IN, HROW), lambda i: (0, 0, 0)),
            pl.BlockSpec((2, FEAT_IN, HROW), lambda i: (0, 0, 0)),
            pl.BlockSpec((2, FEAT_IN, HROW), lambda i: (0, 0, 0)),
            pl.BlockSpec((1, HROW), lambda i: (0, 0)),
        ],
        out_specs=[table_spec] * 6 + [pl.BlockSpec((8, 128), lambda i: (0, 0))],
        out_shape=[table_shape] * 6 + [jax.ShapeDtypeStruct((8, 128), f32)],
    )(x_pad, W, ab, jnp.asarray(_MS), jnp.asarray(_MD), jnp.asarray(_PC),
      jnp.asarray(_ONE))

    m16 = m8[0, :2 * N_HEADS]
    bsum = m16[:N_HEADS] + m16[N_HEADS:]
    b8 = jnp.maximum(bsum, SLOPE * bsum)
    bh = []
    for c in (0, 1):
        bh.append(jnp.concatenate(
            [jnp.repeat(b8[4 * c:4 * c + 4], 18), jnp.zeros((8,), f32)]))

    loop = jnp.arange(N_NODES, dtype=edge_index.dtype)
    fill = jnp.full((E_PAD - E_TOT,), N_NODES, dtype=edge_index.dtype)
    src_i = jnp.concatenate([edge_index[0], loop, fill]).astype(jnp.int32)
    dst_i = jnp.concatenate([edge_index[1], loop, fill]).astype(jnp.int32)

    part = _edge_kernel(s0, d0, h0, bh[0], s1, d1, h1, bh[1], src_i, dst_i)

    return pl.pallas_call(
        _final_body,
        out_shape=jax.ShapeDtypeStruct((N_NODES, N_HEADS * FEAT_OUT), f32),
    )(part, jnp.asarray(_FM), jnp.asarray(_DM), bias.reshape(1, -1),
      gamma.reshape(1, -1), beta.reshape(1, -1))
